# Initial kernel scaffold; baseline (speedup 1.0000x reference)
import jax
import jax.numpy as jnp
from jax.experimental import pallas as pl


def kernel(x, table):
    raise NotImplementedError("write your pallas kernel here")


# speedup vs baseline: 3.7633x; 3.7633x over previous
import functools

import jax
import jax.numpy as jnp
from jax import lax
from jax.experimental import pallas as pl
from jax.experimental.pallas import tpu as pltpu
from jax.experimental.pallas import tpu_sc as plsc

BATCH = 4096
HIST = 50
EMBED = 64
B = BATCH * HIST

NUM_CORES = 2
NUM_SUBCORES = 16
NUM_WORKERS = NUM_CORES * NUM_SUBCORES
B_PER_W = B // NUM_WORKERS
CHUNK = 128
N_CHUNKS = B_PER_W // CHUNK


def _gather_body(table_hbm, idx_hbm, out_hbm, idx_v, rows_v, sem):
    wid = lax.axis_index("s") * NUM_CORES + lax.axis_index("c")
    base = wid * B_PER_W

    def body(i, carry):
        off = base + i * CHUNK
        pltpu.sync_copy(idx_hbm.at[pl.ds(off, CHUNK)], idx_v)
        pltpu.async_copy(table_hbm.at[idx_v], rows_v, sem).wait()
        pltpu.sync_copy(rows_v, out_hbm.at[pl.ds(off, CHUNK)])
        return carry

    lax.fori_loop(0, N_CHUNKS, body, 0)


@jax.jit
def kernel(x, table):
    idx = x.reshape(B).astype(jnp.int32)
    mesh = plsc.VectorSubcoreMesh(core_axis_name="c", subcore_axis_name="s")
    out = pl.kernel(
        _gather_body,
        mesh=mesh,
        out_type=jax.ShapeDtypeStruct((B, EMBED), jnp.float32),
        scratch_types=[
            pltpu.VMEM((CHUNK,), jnp.int32),
            pltpu.VMEM((CHUNK, EMBED), jnp.float32),
            pltpu.SemaphoreType.DMA,
        ],
        compiler_params=pltpu.CompilerParams(use_tc_tiling_on_sc=False),
    )(table, idx)
    return out.reshape(BATCH, HIST, EMBED)


# speedup vs baseline: 4.6248x; 1.2289x over previous
import jax
import jax.numpy as jnp
from jax import lax
from jax.experimental import pallas as pl
from jax.experimental.pallas import tpu as pltpu
from jax.experimental.pallas import tpu_sc as plsc

BATCH = 4096
HIST = 50
EMBED = 64
B = BATCH * HIST
ROW = 128
NROWS = B // ROW
NUM_CORES = 2
NUM_SUBCORES = 16
NUM_WORKERS = NUM_CORES * NUM_SUBCORES
ROWS_PER_W = NROWS // NUM_WORKERS
K = 5
NSUP = ROWS_PER_W // K


def _gather_body(table_hbm, idx_hbm, out_hbm,
                 idx0, idx1, rows0, rows1, gsem0, gsem1):
    wid = lax.axis_index("s") * NUM_CORES + lax.axis_index("c")
    base_row = wid * ROWS_PER_W
    bufs = ((idx0, rows0, gsem0), (idx1, rows1, gsem1))

    def stage(s, b):
        idx_v, rows_v, gsem = bufs[b]
        pltpu.sync_copy(idx_hbm.at[pl.ds(base_row + s * K, K)], idx_v)
        for k in range(K):
            pltpu.async_copy(table_hbm.at[idx_v.at[k]], rows_v.at[k], gsem)

    def drain_writeback(s, b):
        idx_v, rows_v, gsem = bufs[b]
        for k in range(K):
            pltpu.make_async_copy(
                table_hbm.at[idx_v.at[k]], rows_v.at[k], gsem).wait()
        pltpu.sync_copy(rows_v, out_hbm.at[pl.ds(base_row + s * K, K)])

    stage(0, 0)

    def body(i, carry):
        s0 = 2 * i
        stage(s0 + 1, 1)
        drain_writeback(s0, 0)

        @pl.when(i < NSUP // 2 - 1)
        def _():
            stage(s0 + 2, 0)

        drain_writeback(s0 + 1, 1)
        return carry

    lax.fori_loop(0, NSUP // 2, body, 0)


@jax.jit
def kernel(x, table):
    idx = x.reshape(NROWS, ROW).astype(jnp.int32)
    mesh = plsc.VectorSubcoreMesh(core_axis_name="c", subcore_axis_name="s")
    out = pl.kernel(
        _gather_body,
        mesh=mesh,
        out_type=jax.ShapeDtypeStruct((NROWS, ROW, EMBED), jnp.float32),
        scratch_types=[
            pltpu.VMEM((K, ROW), jnp.int32),
            pltpu.VMEM((K, ROW), jnp.int32),
            pltpu.VMEM((K, ROW, EMBED), jnp.float32),
            pltpu.VMEM((K, ROW, EMBED), jnp.float32),
            pltpu.SemaphoreType.DMA,
            pltpu.SemaphoreType.DMA,
        ],
        compiler_params=pltpu.CompilerParams(use_tc_tiling_on_sc=False),
    )(table, idx)
    return out.reshape(BATCH, HIST, EMBED)
